# Initial kernel scaffold; baseline (speedup 1.0000x reference)
import jax
import jax.numpy as jnp
from jax.experimental import pallas as pl


def kernel(logits, target, rel_candidate, rel_wordlist_num, target_num, rel_candidate_num, mask, rel_weight):
    raise NotImplementedError("write your pallas kernel here")


# speedup vs baseline: 177.6917x; 177.6917x over previous
import functools

import jax
import jax.numpy as jnp
from jax import lax
from jax.experimental import pallas as pl
from jax.experimental.pallas import tpu as pltpu
from jax.experimental.pallas import tpu_sc as plsc

N, S, C = 256, 64, 128
NC = 2
NS = 16
NW = NC * NS
NPW = N // NW

_DN = lax.GatherDimensionNumbers(offset_dims=(), collapsed_slice_dims=(0,),
                                 start_index_map=(0,))


def _perm(v, idx):
    return lax.gather(v, idx[:, None], _DN, (1,),
                      mode=lax.GatherScatterMode.PROMISE_IN_BOUNDS)


def _splat(v, i, lane):
    return _perm(v, lane * 0 + i)


def _allmax(v, lane):
    for j in (1, 2, 4, 8):
        v = jnp.maximum(v, _perm(v, lane ^ j))
    return v


def _allsum(v, lane):
    for j in (1, 2, 4, 8):
        v = v + _perm(v, lane ^ j)
    return v


def _sc_body(logits_hbm, target_hbm, rc_hbm, tn_hbm, cn_hbm, mask_hbm,
             rw_hbm, out_hbm, lbuf, rcbuf, tbuf, cnbuf, tnbuf, mbuf, rwbuf,
             obuf):
    wid = lax.axis_index("s") * NC + lax.axis_index("c")
    n0 = wid * NPW
    r0 = n0 * S

    pltpu.sync_copy(target_hbm.at[pl.ds(r0, NPW * S)], tbuf)
    pltpu.sync_copy(cn_hbm.at[pl.ds(r0, NPW * S)], cnbuf)
    pltpu.sync_copy(tn_hbm.at[pl.ds(n0, NPW)], tnbuf.at[pl.ds(0, NPW)])
    pltpu.sync_copy(mask_hbm.at[pl.ds(r0, NPW * S)], mbuf)
    pltpu.sync_copy(rw_hbm.at[pl.ds(r0, NPW * S)], rwbuf)

    lane = lax.iota(jnp.int32, 16)
    tn_all = tnbuf[...]

    def tile_loop(ni, carry):
        acc0, accm0 = carry
        pltpu.sync_copy(logits_hbm.at[pl.ds((n0 + ni) * S * C, S * C)], lbuf)
        pltpu.sync_copy(rc_hbm.at[pl.ds((n0 + ni) * S * C, S * C)], rcbuf)
        tn_v = _splat(tn_all, ni, lane)

        def s_body(s, carry2):
            acc, accm = carry2
            blk = ni * S + (s & ~15)
            sl = s & 15
            t_v = _splat(tbuf[pl.ds(blk, 16)], sl, lane)
            cn_v = _splat(cnbuf[pl.ds(blk, 16)], sl, lane)
            rw_v = _splat(rwbuf[pl.ds(blk, 16)], sl, lane)
            m_v = _splat(mbuf[pl.ds(blk, 16)], sl, lane)
            cn_eff = jnp.where(s < tn_v, cn_v, 0)

            win = jnp.full((16,), -1, jnp.int32)
            val_v = jnp.zeros((16,), jnp.float32)
            for j in range(C // 16):
                gidx = lane + j * 16
                rc16 = rcbuf[pl.ds(s * C + j * 16, 16)]
                l16 = lbuf[pl.ds(s * C + j * 16, 16)]
                m = (rc16 == t_v) & (gidx < cn_eff)
                win = jnp.where(m, gidx, win)
                val_v = jnp.where(m, l16, val_v)
            wmax = _allmax(win, lane)
            sel_w = (win == wmax) & (wmax >= 0)
            val = _allsum(jnp.where(sel_w, val_v, 0.0), lane)
            return acc + val * rw_v, accm + m_v / 16.0

        return lax.fori_loop(0, S, s_body, (acc0, accm0))

    acc, accm = lax.fori_loop(
        0, NPW, tile_loop,
        (jnp.zeros((16,), jnp.float32), jnp.zeros((16,), jnp.float32)))

    obuf[pl.ds(0, 16)] = acc / 16.0
    obuf[pl.ds(16, 16)] = accm
    pltpu.sync_copy(obuf, out_hbm.at[wid])


def _tc_reduce_body(x_ref, o_ref):
    x = x_ref[...]
    col = lax.broadcasted_iota(jnp.int32, x.shape, 1)
    num = jnp.sum(jnp.where(col < 16, x, 0.0))
    den = jnp.sum(jnp.where(col >= 16, x, 0.0))
    o_ref[0, 0] = -num / den


def kernel(logits, target, rel_candidate, rel_wordlist_num, target_num,
           rel_candidate_num, mask, rel_weight):
    del rel_wordlist_num

    mesh = plsc.VectorSubcoreMesh(core_axis_name="c", subcore_axis_name="s")
    sc_kernel = functools.partial(
        pl.kernel,
        out_type=jax.ShapeDtypeStruct((NW, 32), jnp.float32),
        mesh=mesh,
        scratch_types=[
            pltpu.VMEM((S * C,), jnp.float32),
            pltpu.VMEM((S * C,), jnp.int32),
            pltpu.VMEM((NPW * S,), jnp.int32),
            pltpu.VMEM((NPW * S,), jnp.int32),
            pltpu.VMEM((16,), jnp.int32),
            pltpu.VMEM((NPW * S,), jnp.float32),
            pltpu.VMEM((NPW * S,), jnp.float32),
            pltpu.VMEM((32,), jnp.float32),
        ],
    )(_sc_body)
    partials = sc_kernel(
        logits.reshape(N * S * C), target.reshape(N * S),
        rel_candidate.reshape(N * S * C), target_num,
        rel_candidate_num.reshape(N * S), mask.reshape(N * S),
        rel_weight.reshape(N * S))

    out = pl.pallas_call(
        _tc_reduce_body,
        out_specs=pl.BlockSpec(memory_space=pltpu.SMEM),
        out_shape=jax.ShapeDtypeStruct((1, 1), jnp.float32),
    )(partials)
    return out[0, 0]
